# baseline (speedup 1.0000x reference)
import jax
import jax.numpy as jnp
from jax.experimental import pallas as pl
from jax.experimental.pallas import tpu as pltpu

FEA = 768
DK = 64
H = 12
L = 2048
SCALE = 1.0 / 8.0
BL = 512
NB = L // BL


def _head_onehot():
    ci = jax.lax.broadcasted_iota(jnp.int32, (FEA, H), 0)
    hi = jax.lax.broadcasted_iota(jnp.int32, (FEA, H), 1)
    return (ci // DK == hi).astype(jnp.float32)


def _head_onehot_t():
    hi = jax.lax.broadcasted_iota(jnp.int32, (H, FEA), 0)
    ci = jax.lax.broadcasted_iota(jnp.int32, (H, FEA), 1)
    return (ci // DK == hi).astype(jnp.float32)


def _mm_t(x, w):
    return jax.lax.dot_general(x, w, (((1,), (1,)), ((), ())),
                               preferred_element_type=jnp.float32)


def _mm(x, w):
    return jax.lax.dot_general(x, w, (((1,), (0,)), ((), ())),
                               preferred_element_type=jnp.float32)


def _body(qx, kx, vx, wq, wk, wv, wo, bq, bk, bv, bo, out, Qs, Ks, Vs, sall_s):
    p = pl.program_id(0)
    j = pl.program_id(1)
    base = j * BL

    @pl.when(p == 0)
    def _proj():
        Qs[pl.ds(base, BL), :] = (_mm_t(qx[...], wq[...]) + bq[...]) * SCALE
        Ks[pl.ds(base, BL), :] = _mm_t(kx[...], wk[...]) + bk[...]
        vv = _mm_t(vx[...], wv[...]) + bv[...]
        Vs[pl.ds(base, BL), :] = vv
        part = jnp.sum(vv, axis=0, keepdims=True)

        @pl.when(j == 0)
        def _():
            sall_s[...] = part

        @pl.when(j > 0)
        def _():
            sall_s[...] += part

    @pl.when(p == 1)
    def _attn():
        E = _head_onehot()
        ET = _head_onehot_t()

        Qb = Qs[pl.ds(base, BL), :]
        k0 = Ks[0:1, :]
        kL = Ks[L - 1:L, :]
        v0 = Vs[0:1, :]
        vL = Vs[L - 1:L, :]
        sall = sall_s[...]

        kblk = Ks[pl.ds(base, BL), :]
        vblk = Vs[pl.ds(base, BL), :]
        kprev = Ks[pl.ds(jnp.maximum(base - 1, 0), 1), :]
        knext = Ks[pl.ds(jnp.minimum(base + BL, L - 1), 1), :]
        vprev = Vs[pl.ds(jnp.maximum(base - 1, 0), 1), :]
        vnext = Vs[pl.ds(jnp.minimum(base + BL, L - 1), 1), :]
        km1 = jnp.concatenate([kprev, kblk[:BL - 1, :]], axis=0)
        kp1 = jnp.concatenate([kblk[1:, :], knext], axis=0)
        vm1 = jnp.concatenate([vprev, vblk[:BL - 1, :]], axis=0)
        vp1 = jnp.concatenate([vblk[1:, :], vnext], axis=0)

        x0 = jnp.exp(_mm(Qb * k0, E))
        xL = jnp.exp(_mm(Qb * kL, E))
        xd = jnp.exp(_mm(Qb * kblk, E))
        xsub = jnp.exp(_mm(Qb * km1, E))
        xsup = jnp.exp(_mm(Qb * kp1, E))

        gi = base + jax.lax.broadcasted_iota(jnp.int32, (BL, 1), 0)
        msub = (gi != 1).astype(jnp.float32)
        msup = (gi != L - 2).astype(jnp.float32)

        denom = (x0 + xL + xd + msub * xsub + msup * xsup
                 + (jnp.float32(L - 3) - msub - msup))

        num = (sall
               + _mm(x0 - 1.0, ET) * v0
               + _mm(xL - 1.0, ET) * vL
               + _mm(xd - 1.0, ET) * vblk
               + _mm(msub * (xsub - 1.0), ET) * vm1
               + _mm(msup * (xsup - 1.0), ET) * vp1)
        z = num / _mm(denom, ET)

        out[...] = _mm_t(z.astype(jnp.bfloat16), wo[...]) + bo[...]

        @pl.when(j == 0)
        def _():
            s0 = _mm(Ks[...] * Qb[0:1, :], E)
            a0 = jnp.exp(s0 - jnp.max(s0, axis=0, keepdims=True))
            alpha0 = a0 / jnp.sum(a0, axis=0, keepdims=True)
            z0 = jnp.sum(_mm(alpha0, ET) * Vs[...], axis=0, keepdims=True)
            out[0:1, :] = _mm_t(z0.astype(jnp.bfloat16), wo[...]) + bo[...]

        @pl.when(j == NB - 1)
        def _():
            sL = _mm(Ks[...] * Qb[BL - 1:BL, :], E)
            aL = jnp.exp(sL - jnp.max(sL, axis=0, keepdims=True))
            alphaL = aL / jnp.sum(aL, axis=0, keepdims=True)
            zL = jnp.sum(_mm(alphaL, ET) * Vs[...], axis=0, keepdims=True)
            out[BL - 1:BL, :] = _mm_t(zL.astype(jnp.bfloat16), wo[...]) + bo[...]


def kernel(qx, kx, vx, WQ_w, WQ_b, WK_w, WK_b, WV_w, WV_b, WO_w, WO_b):
    q2 = qx.reshape(L, FEA).astype(jnp.bfloat16)
    k2 = kx.reshape(L, FEA).astype(jnp.bfloat16)
    v2 = vx.reshape(L, FEA).astype(jnp.bfloat16)
    wqh = WQ_w.astype(jnp.bfloat16)
    wkh = WK_w.astype(jnp.bfloat16)
    wvh = WV_w.astype(jnp.bfloat16)
    woh = WO_w.astype(jnp.bfloat16)
    bq = WQ_b.reshape(1, FEA)
    bk = WK_b.reshape(1, FEA)
    bv = WV_b.reshape(1, FEA)
    bo = WO_b.reshape(1, FEA)

    in_blk = pl.BlockSpec((BL, FEA), lambda p, j: (j * (1 - p), 0))
    full_w = pl.BlockSpec((FEA, FEA), lambda p, j: (0, 0))
    full_b = pl.BlockSpec((1, FEA), lambda p, j: (0, 0))
    out_blk = pl.BlockSpec((BL, FEA), lambda p, j: (j * p, 0))

    out = pl.pallas_call(
        _body,
        grid=(2, NB),
        in_specs=[in_blk, in_blk, in_blk, full_w, full_w, full_w, full_w,
                  full_b, full_b, full_b, full_b],
        out_specs=out_blk,
        out_shape=jax.ShapeDtypeStruct((L, FEA), jnp.float32),
        compiler_params=pltpu.CompilerParams(vmem_limit_bytes=100 * 1024 * 1024),
        scratch_shapes=[
            pltpu.VMEM((L, FEA), jnp.float32),
            pltpu.VMEM((L, FEA), jnp.float32),
            pltpu.VMEM((L, FEA), jnp.float32),
            pltpu.VMEM((1, FEA), jnp.float32),
        ],
    )(q2, k2, v2, wqh, wkh, wvh, woh, bq, bk, bv, bo)

    return out.reshape(1, L, FEA)


# speedup vs baseline: 1.3457x; 1.3457x over previous
import jax
import jax.numpy as jnp
from jax.experimental import pallas as pl
from jax.experimental.pallas import tpu as pltpu

FEA = 768
DK = 64
H = 12
L = 2048
SCALE = 1.0 / 8.0
BL = 512
NB = L // BL


def _head_onehot():
    ci = jax.lax.broadcasted_iota(jnp.int32, (FEA, H), 0)
    hi = jax.lax.broadcasted_iota(jnp.int32, (FEA, H), 1)
    return (ci // DK == hi).astype(jnp.float32)


def _head_onehot_t():
    hi = jax.lax.broadcasted_iota(jnp.int32, (H, FEA), 0)
    ci = jax.lax.broadcasted_iota(jnp.int32, (H, FEA), 1)
    return (ci // DK == hi).astype(jnp.float32)


def _mm_t(x, w):
    return jax.lax.dot_general(x, w, (((1,), (1,)), ((), ())),
                               preferred_element_type=jnp.float32)


def _mm(x, w):
    return jax.lax.dot_general(x, w, (((1,), (0,)), ((), ())),
                               preferred_element_type=jnp.float32)


def _body(qx, kx, vx, wq, wk, wv, wo, bq, bk, bv, bo, out, Qs, Ks, Vs, sall_s):
    p = pl.program_id(0)
    j = pl.program_id(1)
    base = j * BL

    @pl.when(p == 0)
    def _proj():
        Qs[pl.ds(base, BL), :] = (_mm_t(qx[...], wq[...]) + bq[...]) * SCALE
        Ks[pl.ds(base, BL), :] = _mm_t(kx[...], wk[...]) + bk[...]
        vv = _mm_t(vx[...], wv[...]) + bv[...]
        Vs[pl.ds(base, BL), :] = vv
        part = jnp.sum(vv, axis=0, keepdims=True)

        @pl.when(j == 0)
        def _():
            sall_s[...] = part

        @pl.when(j > 0)
        def _():
            sall_s[...] += part

    @pl.when(p == 1)
    def _attn():
        E = _head_onehot()
        ET = _head_onehot_t()

        Qb = Qs[pl.ds(base, BL), :]
        k0 = Ks[0:1, :]
        kL = Ks[L - 1:L, :]
        v0 = Vs[0:1, :]
        vL = Vs[L - 1:L, :]
        sall = sall_s[...]

        kblk = Ks[pl.ds(base, BL), :]
        vblk = Vs[pl.ds(base, BL), :]
        kprev = Ks[pl.ds(jnp.maximum(base - 1, 0), 1), :]
        knext = Ks[pl.ds(jnp.minimum(base + BL, L - 1), 1), :]
        vprev = Vs[pl.ds(jnp.maximum(base - 1, 0), 1), :]
        vnext = Vs[pl.ds(jnp.minimum(base + BL, L - 1), 1), :]
        km1 = jnp.concatenate([kprev, kblk[:BL - 1, :]], axis=0)
        kp1 = jnp.concatenate([kblk[1:, :], knext], axis=0)
        vm1 = jnp.concatenate([vprev, vblk[:BL - 1, :]], axis=0)
        vp1 = jnp.concatenate([vblk[1:, :], vnext], axis=0)

        x0 = jnp.exp(_mm(Qb * k0, E))
        xL = jnp.exp(_mm(Qb * kL, E))
        xd = jnp.exp(_mm(Qb * kblk, E))
        xsub = jnp.exp(_mm(Qb * km1, E))
        xsup = jnp.exp(_mm(Qb * kp1, E))

        gi = base + jax.lax.broadcasted_iota(jnp.int32, (BL, 1), 0)
        msub = (gi != 1).astype(jnp.float32)
        msup = (gi != L - 2).astype(jnp.float32)

        denom = (x0 + xL + xd + msub * xsub + msup * xsup
                 + (jnp.float32(L - 3) - msub - msup))

        num = (sall
               + _mm(x0 - 1.0, ET) * v0
               + _mm(xL - 1.0, ET) * vL
               + _mm(xd - 1.0, ET) * vblk
               + _mm(msub * (xsub - 1.0), ET) * vm1
               + _mm(msup * (xsup - 1.0), ET) * vp1)
        z = num / _mm(denom, ET)

        out[...] = _mm_t(z, wo[...]) + bo[...]

        @pl.when(j == 0)
        def _():
            s0 = _mm(Ks[...] * Qb[0:1, :], E)
            a0 = jnp.exp(s0 - jnp.max(s0, axis=0, keepdims=True))
            alpha0 = a0 / jnp.sum(a0, axis=0, keepdims=True)
            z0 = jnp.sum(_mm(alpha0, ET) * Vs[...], axis=0, keepdims=True)
            out[0:1, :] = _mm_t(z0, wo[...]) + bo[...]

        @pl.when(j == NB - 1)
        def _():
            sL = _mm(Ks[...] * Qb[BL - 1:BL, :], E)
            aL = jnp.exp(sL - jnp.max(sL, axis=0, keepdims=True))
            alphaL = aL / jnp.sum(aL, axis=0, keepdims=True)
            zL = jnp.sum(_mm(alphaL, ET) * Vs[...], axis=0, keepdims=True)
            out[BL - 1:BL, :] = _mm_t(zL, wo[...]) + bo[...]


def kernel(qx, kx, vx, WQ_w, WQ_b, WK_w, WK_b, WV_w, WV_b, WO_w, WO_b):
    q2 = qx.reshape(L, FEA)
    k2 = kx.reshape(L, FEA)
    v2 = vx.reshape(L, FEA)
    bq = WQ_b.reshape(1, FEA)
    bk = WK_b.reshape(1, FEA)
    bv = WV_b.reshape(1, FEA)
    bo = WO_b.reshape(1, FEA)

    in_blk = pl.BlockSpec((BL, FEA), lambda p, j: (j * (1 - p), 0))
    full_w = pl.BlockSpec((FEA, FEA), lambda p, j: (0, 0))
    full_b = pl.BlockSpec((1, FEA), lambda p, j: (0, 0))
    out_blk = pl.BlockSpec((BL, FEA), lambda p, j: (j * p, 0))

    out = pl.pallas_call(
        _body,
        grid=(2, NB),
        in_specs=[in_blk, in_blk, in_blk, full_w, full_w, full_w, full_w,
                  full_b, full_b, full_b, full_b],
        out_specs=out_blk,
        out_shape=jax.ShapeDtypeStruct((L, FEA), jnp.float32),
        compiler_params=pltpu.CompilerParams(vmem_limit_bytes=100 * 1024 * 1024),
        scratch_shapes=[
            pltpu.VMEM((L, FEA), jnp.float32),
            pltpu.VMEM((L, FEA), jnp.float32),
            pltpu.VMEM((L, FEA), jnp.float32),
            pltpu.VMEM((1, FEA), jnp.float32),
        ],
    )(q2, k2, v2, WQ_w, WK_w, WV_w, WO_w, bq, bk, bv, bo)

    return out.reshape(1, L, FEA)


# speedup vs baseline: 1.4445x; 1.0734x over previous
import jax
import jax.numpy as jnp
from jax.experimental import pallas as pl
from jax.experimental.pallas import tpu as pltpu

FEA = 768
DK = 64
H = 12
L = 2048
SCALE = 1.0 / 8.0
BL = 512
NB = L // BL


def _head_onehot():
    ci = jax.lax.broadcasted_iota(jnp.int32, (FEA, H), 0)
    hi = jax.lax.broadcasted_iota(jnp.int32, (FEA, H), 1)
    return (ci // DK == hi).astype(jnp.float32)


def _head_onehot_t():
    hi = jax.lax.broadcasted_iota(jnp.int32, (H, FEA), 0)
    ci = jax.lax.broadcasted_iota(jnp.int32, (H, FEA), 1)
    return (ci // DK == hi).astype(jnp.float32)


def _mm_t(x, w):
    return jax.lax.dot_general(x, w, (((1,), (1,)), ((), ())),
                               preferred_element_type=jnp.float32)


def _mm(x, w):
    return jax.lax.dot_general(x, w, (((1,), (0,)), ((), ())),
                               preferred_element_type=jnp.float32)


def _global_row(qrow, ET, Ks, Vs, wo, bo):
    G = ET * qrow
    s = _mm_t(G, Ks[...])
    a = jnp.exp(s - jnp.max(s, axis=1, keepdims=True))
    alpha = a / jnp.sum(a, axis=1, keepdims=True)
    Z = _mm(alpha, Vs[...])
    z = jnp.sum(ET * Z, axis=0, keepdims=True)
    return _mm_t(z, wo) + bo


def _body(qx, kx, vx, wq, wk, wv, wo, bq, bk, bv, bo, out, Ks, Vs, sall_s):
    p = pl.program_id(0)
    j = pl.program_id(1)
    base = j * BL

    @pl.when(p == 0)
    def _proj():
        Ks[pl.ds(base, BL), :] = _mm_t(kx[...], wk[...]) + bk[...]
        vv = _mm_t(vx[...], wv[...]) + bv[...]
        Vs[pl.ds(base, BL), :] = vv
        part = jnp.sum(vv, axis=0, keepdims=True)

        @pl.when(j == 0)
        def _():
            sall_s[...] = part

        @pl.when(j > 0)
        def _():
            sall_s[...] += part

    @pl.when(p == 1)
    def _attn():
        E = _head_onehot()
        ET = _head_onehot_t()

        Qb = (_mm_t(qx[...], wq[...]) + bq[...]) * SCALE

        k0 = Ks[0:1, :]
        kL = Ks[L - 1:L, :]
        v0 = Vs[0:1, :]
        vL = Vs[L - 1:L, :]
        sall = sall_s[...]

        kblk = Ks[pl.ds(base, BL), :]
        vblk = Vs[pl.ds(base, BL), :]
        kprev = Ks[pl.ds(jnp.maximum(base - 1, 0), 1), :]
        knext = Ks[pl.ds(jnp.minimum(base + BL, L - 1), 1), :]
        vprev = Vs[pl.ds(jnp.maximum(base - 1, 0), 1), :]
        vnext = Vs[pl.ds(jnp.minimum(base + BL, L - 1), 1), :]
        km1 = jnp.concatenate([kprev, kblk[:BL - 1, :]], axis=0)
        kp1 = jnp.concatenate([kblk[1:, :], knext], axis=0)
        vm1 = jnp.concatenate([vprev, vblk[:BL - 1, :]], axis=0)
        vp1 = jnp.concatenate([vblk[1:, :], vnext], axis=0)

        G2 = jnp.concatenate([ET * k0, ET * kL], axis=0)
        e0L = _mm_t(Qb, G2)
        x0 = jnp.exp(e0L[:, :H])
        xL = jnp.exp(e0L[:, H:])
        xd = jnp.exp(_mm(Qb * kblk, E))
        xsub = jnp.exp(_mm(Qb * km1, E))
        xsup = jnp.exp(_mm(Qb * kp1, E))

        gi = base + jax.lax.broadcasted_iota(jnp.int32, (BL, 1), 0)
        msub = (gi != 1).astype(jnp.float32)
        msup = (gi != L - 2).astype(jnp.float32)

        denom = (x0 + xL + xd + msub * xsub + msup * xsup
                 + (jnp.float32(L - 3) - msub - msup))
        recip = 1.0 / denom

        X3 = jnp.concatenate(
            [recip, recip * (x0 - 1.0), recip * (xL - 1.0)], axis=1)
        W3 = jnp.concatenate([ET * sall, ET * v0, ET * vL], axis=0)
        z = (_mm(X3, W3)
             + _mm(recip * (xd - 1.0), ET) * vblk
             + _mm(recip * msub * (xsub - 1.0), ET) * vm1
             + _mm(recip * msup * (xsup - 1.0), ET) * vp1)

        out[...] = _mm_t(z, wo[...]) + bo[...]

        @pl.when(j == 0)
        def _():
            out[0:1, :] = _global_row(Qb[0:1, :], ET, Ks, Vs, wo[...], bo[...])

        @pl.when(j == NB - 1)
        def _():
            out[BL - 1:BL, :] = _global_row(Qb[BL - 1:BL, :], ET, Ks, Vs,
                                            wo[...], bo[...])


def kernel(qx, kx, vx, WQ_w, WQ_b, WK_w, WK_b, WV_w, WV_b, WO_w, WO_b):
    q2 = qx.reshape(L, FEA)
    k2 = kx.reshape(L, FEA)
    v2 = vx.reshape(L, FEA)
    bq = WQ_b.reshape(1, FEA)
    bk = WK_b.reshape(1, FEA)
    bv = WV_b.reshape(1, FEA)
    bo = WO_b.reshape(1, FEA)

    q_blk = pl.BlockSpec((BL, FEA), lambda p, j: (j, 0))
    kv_blk = pl.BlockSpec((BL, FEA), lambda p, j: (j * (1 - p), 0))
    full_w = pl.BlockSpec((FEA, FEA), lambda p, j: (0, 0))
    full_b = pl.BlockSpec((1, FEA), lambda p, j: (0, 0))
    out_blk = pl.BlockSpec((BL, FEA), lambda p, j: (j * p, 0))

    out = pl.pallas_call(
        _body,
        grid=(2, NB),
        in_specs=[q_blk, kv_blk, kv_blk, full_w, full_w, full_w, full_w,
                  full_b, full_b, full_b, full_b],
        out_specs=out_blk,
        out_shape=jax.ShapeDtypeStruct((L, FEA), jnp.float32),
        compiler_params=pltpu.CompilerParams(vmem_limit_bytes=100 * 1024 * 1024),
        scratch_shapes=[
            pltpu.VMEM((L, FEA), jnp.float32),
            pltpu.VMEM((L, FEA), jnp.float32),
            pltpu.VMEM((1, FEA), jnp.float32),
        ],
    )(q2, k2, v2, WQ_w, WK_w, WV_w, WO_w, bq, bk, bv, bo)

    return out.reshape(1, L, FEA)


# speedup vs baseline: 1.4576x; 1.0091x over previous
import jax
import jax.numpy as jnp
from jax.experimental import pallas as pl
from jax.experimental.pallas import tpu as pltpu

FEA = 768
DK = 64
H = 12
L = 2048
SCALE = 1.0 / 8.0
BL = 512
NB = L // BL


def _head_onehot():
    ci = jax.lax.broadcasted_iota(jnp.int32, (FEA, H), 0)
    hi = jax.lax.broadcasted_iota(jnp.int32, (FEA, H), 1)
    return (ci // DK == hi).astype(jnp.float32)


def _head_onehot_t():
    hi = jax.lax.broadcasted_iota(jnp.int32, (H, FEA), 0)
    ci = jax.lax.broadcasted_iota(jnp.int32, (H, FEA), 1)
    return (ci // DK == hi).astype(jnp.float32)


def _mm_t(x, w):
    return jax.lax.dot_general(x, w, (((1,), (1,)), ((), ())),
                               preferred_element_type=jnp.float32)


def _mm(x, w):
    return jax.lax.dot_general(x, w, (((1,), (0,)), ((), ())),
                               preferred_element_type=jnp.float32)


def _global_row(qrow, ET, Ks, Vs, wo, bo):
    G = (ET * qrow).astype(jnp.bfloat16)
    s = _mm_t(G, Ks[...])
    a = jnp.exp(s - jnp.max(s, axis=1, keepdims=True))
    alpha = (a / jnp.sum(a, axis=1, keepdims=True)).astype(jnp.bfloat16)
    Z = _mm(alpha, Vs[...])
    z = jnp.sum(ET * Z, axis=0, keepdims=True)
    return _mm_t(z.astype(jnp.bfloat16), wo) + bo


def _body(qx, kx, vx, wq, wk, wv, wo, bq, bk, bv, bo, out, Ks, Vs, sall_s):
    p = pl.program_id(0)
    j = pl.program_id(1)
    base = j * BL

    @pl.when(p == 0)
    def _proj():
        kxb = kx[...].astype(jnp.bfloat16)
        vxb = vx[...].astype(jnp.bfloat16)
        wkb = wk[...].astype(jnp.bfloat16)
        wvb = wv[...].astype(jnp.bfloat16)
        Ks[pl.ds(base, BL), :] = (_mm_t(kxb, wkb) + bk[...]).astype(jnp.bfloat16)
        vv = _mm_t(vxb, wvb) + bv[...]
        Vs[pl.ds(base, BL), :] = vv.astype(jnp.bfloat16)
        part = jnp.sum(vv, axis=0, keepdims=True)

        @pl.when(j == 0)
        def _():
            sall_s[...] = part

        @pl.when(j > 0)
        def _():
            sall_s[...] += part

    @pl.when(p == 1)
    def _attn():
        E = _head_onehot()
        ET = _head_onehot_t()

        qxb = qx[...].astype(jnp.bfloat16)
        wqb = wq[...].astype(jnp.bfloat16)
        wob = wo[...].astype(jnp.bfloat16)
        Qb = (_mm_t(qxb, wqb) + bq[...]) * SCALE

        k0 = Ks[0:8, :][0:1].astype(jnp.float32)
        kL = Ks[L - 8:L, :][7:8].astype(jnp.float32)
        v0 = Vs[0:8, :][0:1].astype(jnp.float32)
        vL = Vs[L - 8:L, :][7:8].astype(jnp.float32)
        sall = sall_s[...]

        kblk = Ks[pl.ds(base, BL), :].astype(jnp.float32)
        vblk = Vs[pl.ds(base, BL), :].astype(jnp.float32)
        prev_w = jnp.maximum(j * (BL // 8) - 1, 0) * 8
        next_w = jnp.minimum((j + 1) * (BL // 8), L // 8 - 1) * 8
        kprev = Ks[pl.ds(prev_w, 8), :][7:8].astype(jnp.float32)
        knext = Ks[pl.ds(next_w, 8), :][0:1].astype(jnp.float32)
        vprev = Vs[pl.ds(prev_w, 8), :][7:8].astype(jnp.float32)
        vnext = Vs[pl.ds(next_w, 8), :][0:1].astype(jnp.float32)
        km1 = jnp.concatenate([kprev, kblk[:BL - 1, :]], axis=0)
        kp1 = jnp.concatenate([kblk[1:, :], knext], axis=0)
        vm1 = jnp.concatenate([vprev, vblk[:BL - 1, :]], axis=0)
        vp1 = jnp.concatenate([vblk[1:, :], vnext], axis=0)

        G2 = jnp.concatenate([ET * k0, ET * kL], axis=0)
        e0L = _mm_t(Qb, G2)
        x0 = jnp.exp(e0L[:, :H])
        xL = jnp.exp(e0L[:, H:])
        xd = jnp.exp(_mm(Qb * kblk, E))
        xsub = jnp.exp(_mm(Qb * km1, E))
        xsup = jnp.exp(_mm(Qb * kp1, E))

        gi = base + jax.lax.broadcasted_iota(jnp.int32, (BL, 1), 0)
        msub = (gi != 1).astype(jnp.float32)
        msup = (gi != L - 2).astype(jnp.float32)

        denom = (x0 + xL + xd + msub * xsub + msup * xsup
                 + (jnp.float32(L - 3) - msub - msup))
        recip = 1.0 / denom

        X3 = jnp.concatenate(
            [recip, recip * (x0 - 1.0), recip * (xL - 1.0)], axis=1)
        W3 = jnp.concatenate([ET * sall, ET * v0, ET * vL], axis=0)
        z = (_mm(X3, W3)
             + _mm(recip * (xd - 1.0), ET) * vblk
             + _mm(recip * msub * (xsub - 1.0), ET) * vm1
             + _mm(recip * msup * (xsup - 1.0), ET) * vp1)

        out[...] = _mm_t(z.astype(jnp.bfloat16), wob) + bo[...]

        @pl.when(j == 0)
        def _():
            out[0:1, :] = _global_row(Qb[0:1, :], ET, Ks, Vs, wob, bo[...])

        @pl.when(j == NB - 1)
        def _():
            out[BL - 1:BL, :] = _global_row(Qb[BL - 1:BL, :], ET, Ks, Vs,
                                            wob, bo[...])


def kernel(qx, kx, vx, WQ_w, WQ_b, WK_w, WK_b, WV_w, WV_b, WO_w, WO_b):
    q2 = qx.reshape(L, FEA)
    k2 = kx.reshape(L, FEA)
    v2 = vx.reshape(L, FEA)
    bq = WQ_b.reshape(1, FEA)
    bk = WK_b.reshape(1, FEA)
    bv = WV_b.reshape(1, FEA)
    bo = WO_b.reshape(1, FEA)

    q_blk = pl.BlockSpec((BL, FEA), lambda p, j: (j, 0))
    kv_blk = pl.BlockSpec((BL, FEA), lambda p, j: (j * (1 - p), 0))
    full_w = pl.BlockSpec((FEA, FEA), lambda p, j: (0, 0))
    full_b = pl.BlockSpec((1, FEA), lambda p, j: (0, 0))
    out_blk = pl.BlockSpec((BL, FEA), lambda p, j: (j * p, 0))

    out = pl.pallas_call(
        _body,
        grid=(2, NB),
        in_specs=[q_blk, kv_blk, kv_blk, full_w, full_w, full_w, full_w,
                  full_b, full_b, full_b, full_b],
        out_specs=out_blk,
        out_shape=jax.ShapeDtypeStruct((L, FEA), jnp.float32),
        compiler_params=pltpu.CompilerParams(vmem_limit_bytes=100 * 1024 * 1024),
        scratch_shapes=[
            pltpu.VMEM((L, FEA), jnp.bfloat16),
            pltpu.VMEM((L, FEA), jnp.bfloat16),
            pltpu.VMEM((1, FEA), jnp.float32),
        ],
    )(q2, k2, v2, WQ_w, WK_w, WV_w, WO_w, bq, bk, bv, bo)

    return out.reshape(1, L, FEA)
